# baseline (speedup 1.0000x reference)
import functools

import jax
import jax.numpy as jnp
from jax import lax
from jax.experimental import pallas as pl
from jax.experimental.pallas import tpu as pltpu
from jax.experimental.pallas import tpu_sc as plsc

N_NODES = 10000
N_EDGES = 160000
D = 256
DH = 128
NG = 64

EB = 2000
NB = 2000
SCB = 80
N_SUB = 16
EPT = N_EDGES // N_SUB
NWR = 10
RPT = N_NODES // NWR
ZROWS = 40
WROWS = 200

_f32 = jnp.float32


def _edge_body(ea_ref, we_ref, be_ref, out_ref):
    acc = jnp.dot(ea_ref[...], we_ref[...], preferred_element_type=_f32)
    eb = jnp.maximum(acc + be_ref[...], 0.0)
    words = []
    for g in range(DH // 32):
        a = eb[:, g * 32:g * 32 + 16].astype(jnp.bfloat16).astype(_f32)
        b = eb[:, g * 32 + 16:g * 32 + 32].astype(jnp.bfloat16).astype(_f32)
        ai = lax.bitcast_convert_type(a, jnp.int32)
        bi = lax.bitcast_convert_type(b, jnp.int32)
        words.append(lax.bitwise_or(
            lax.bitwise_and(bi, jnp.int32(-65536)),
            lax.shift_right_logical(ai, 16)))
    out_ref[...] = jnp.concatenate(words, axis=1)


def _edge_mlp(edge_attr, We, be2):
    nblk = N_EDGES // EB
    return pl.pallas_call(
        _edge_body,
        grid=(2, nblk),
        in_specs=[
            pl.BlockSpec((EB, 16), lambda c, j: (j, 0)),
            pl.BlockSpec((16, DH), lambda c, j: (0, c)),
            pl.BlockSpec((1, DH), lambda c, j: (0, c)),
        ],
        out_specs=pl.BlockSpec((EB, DH // 2),
                               lambda c, j, _n=nblk: (c * _n + j, 0)),
        out_shape=jax.ShapeDtypeStruct((2 * N_EDGES, DH // 2), jnp.int32),
    )(edge_attr, We, be2)


_sc_mesh = plsc.VectorSubcoreMesh(core_axis_name="c", subcore_axis_name="s")


IB = 25
NBATCH = EPT // (IB * SCB)


@functools.partial(
    pl.kernel,
    out_type=jax.ShapeDtypeStruct((2 * N_NODES, DH), _f32),
    mesh=_sc_mesh,
    scratch_types=[
        pltpu.VMEM((IB * SCB,), jnp.int32),
        pltpu.VMEM((IB * SCB,), jnp.int32),
        pltpu.VMEM((SCB,), jnp.int32),
        pltpu.VMEM((SCB,), jnp.int32),
        pltpu.VMEM((SCB, DH), _f32),
        pltpu.VMEM((SCB, DH), _f32),
        pltpu.VMEM((SCB, DH // 2), jnp.int32),
        pltpu.VMEM((SCB, DH // 2), jnp.int32),
        pltpu.VMEM((ZROWS, DH), _f32),
        pltpu.VMEM_SHARED((N_NODES, DH), _f32),
        pltpu.SemaphoreType.DMA,
        pltpu.SemaphoreType.DMA,
    ],
)
def _sc_edge(h_hbm, e_hbm, src_hbm, dst_hbm, out_hbm,
             idx_sbig, idx_dbig, idx_d0, idx_d1, g0, g1, e0, e1,
             zero_v, shared, sem0, sem1):
    c = lax.axis_index("c")
    s = lax.axis_index("s")
    idx_d = (idx_d0, idx_d1)
    gath = (g0, g1)
    e_v = (e0, e1)
    sem = (sem0, sem1)

    def zrow(r, carry):
        for t in range(DH // 16):
            zero_v[r, pl.ds(t * 16, 16)] = jnp.zeros((16,), _f32)
        return carry

    @pl.when(s < NWR)
    def _zero():
        lax.fori_loop(0, ZROWS, zrow, 0)
        for i in range(RPT // ZROWS):
            pltpu.sync_copy(zero_v, shared.at[pl.ds(s * RPT + i * ZROWS, ZROWS)])

    plsc.subcore_barrier()

    ebase = s * EPT

    def batch(t, carry):
        boff = ebase + t * (IB * SCB)
        pltpu.sync_copy(src_hbm.at[pl.ds(boff, IB * SCB)], idx_sbig)
        pltpu.sync_copy(dst_hbm.at[pl.ds(boff, IB * SCB)], idx_dbig)

        def adj(i, ac):
            sl = pl.ds(i * 16, 16)
            idx_sbig[sl] = idx_sbig[sl] + c * N_NODES
            return ac

        lax.fori_loop(0, IB * SCB // 16, adj, 0)

        def start(k, b):
            for t in range(SCB // 16):
                idx_d[b][pl.ds(t * 16, 16)] = idx_dbig[pl.ds(k * SCB + t * 16, 16)]
            pltpu.async_copy(h_hbm.at[idx_sbig.at[pl.ds(k * SCB, SCB)]],
                             gath[b], sem[b])
            pltpu.async_copy(
                e_hbm.at[pl.ds(c * N_EDGES + boff + k * SCB, SCB)],
                e_v[b], sem[b])

        def finish(k, b):
            pltpu.make_async_copy(h_hbm.at[idx_sbig.at[pl.ds(k * SCB, SCB)]],
                                  gath[b], sem[b]).wait()
            pltpu.make_async_copy(
                e_hbm.at[pl.ds(c * N_EDGES + boff + k * SCB, SCB)],
                e_v[b], sem[b]).wait()

            @plsc.parallel_loop(0, SCB, unroll=4)
            def _row(r):
                for g in range(DH // 32):
                    w = e_v[b][r, pl.ds(g * 16, 16)]
                    lo = lax.bitcast_convert_type(
                        lax.shift_left(w, 16), _f32)
                    hi = lax.bitcast_convert_type(
                        lax.bitwise_and(w, jnp.int32(-65536)), _f32)
                    sl0 = pl.ds(g * 32, 16)
                    sl1 = pl.ds(g * 32 + 16, 16)
                    gath[b][r, sl0] = jnp.maximum(gath[b][r, sl0] + lo, 0.0)
                    gath[b][r, sl1] = jnp.maximum(gath[b][r, sl1] + hi, 0.0)

            pltpu.sync_copy(gath[b], shared.at[idx_d[b]], add=True)

        start(0, 0)
        for k in range(IB - 1):
            start(k + 1, (k + 1) % 2)
            finish(k, k % 2)
        finish(IB - 1, (IB - 1) % 2)
        return carry

    lax.fori_loop(0, NBATCH, batch, 0)
    plsc.subcore_barrier()

    @pl.when(s < NWR)
    def _writeout():
        for i in range(RPT // WROWS):
            rb = s * RPT + i * WROWS
            pltpu.sync_copy(shared.at[pl.ds(rb, WROWS)],
                            out_hbm.at[pl.ds(c * N_NODES + rb, WROWS)])


def _node_body(h_ref, agg_ref, w_ref, b_ref, out_ref):
    hp0 = h_ref[0] + agg_ref[0]
    hp1 = h_ref[1] + agg_ref[1]
    acc = jnp.dot(hp0, w_ref[0:DH, :], preferred_element_type=_f32)
    acc += jnp.dot(hp1, w_ref[DH:2 * DH, :], preferred_element_type=_f32)
    out_ref[...] = jnp.maximum(acc + b_ref[...], 0.0)


def _node_update(h3, agg3, W, b2):
    nblk = N_NODES // NB
    return pl.pallas_call(
        _node_body,
        grid=(2, nblk),
        in_specs=[
            pl.BlockSpec((2, NB, DH), lambda c2, j: (0, j, 0)),
            pl.BlockSpec((2, NB, DH), lambda c2, j: (0, j, 0)),
            pl.BlockSpec((D, DH), lambda c2, j: (0, c2)),
            pl.BlockSpec((1, DH), lambda c2, j: (0, c2)),
        ],
        out_specs=pl.BlockSpec((NB, DH), lambda c2, j, _n=nblk: (c2 * _n + j, 0)),
        out_shape=jax.ShapeDtypeStruct((2 * N_NODES, DH), _f32),
    )(h3, agg3, W, b2)


def _head_body(h_ref, bat_ref, wh_ref, bh_ref, wv_ref, bv_ref,
               log_ref, val_ref, s0, s1, cnt):
    j = pl.program_id(0)
    nb = pl.num_programs(0)
    h0 = h_ref[0]
    h1 = h_ref[1]
    lg = jnp.dot(h0, wh_ref[0:DH, :], preferred_element_type=_f32)
    lg += jnp.dot(h1, wh_ref[DH:2 * DH, :], preferred_element_type=_f32)
    log_ref[...] = lg + bh_ref[...]

    onehot = (bat_ref[...] == lax.broadcasted_iota(jnp.int32, (1, NG), 1))
    onehot = onehot.astype(_f32)
    dn = (((0,), (0,)), ((), ()))
    ps0 = lax.dot_general(onehot, h0, dn, preferred_element_type=_f32)
    ps1 = lax.dot_general(onehot, h1, dn, preferred_element_type=_f32)
    pc = lax.dot_general(onehot, jnp.ones((NB, 1), _f32), dn,
                         preferred_element_type=_f32)

    @pl.when(j == 0)
    def _init():
        s0[...] = ps0
        s1[...] = ps1
        cnt[...] = pc

    @pl.when(j > 0)
    def _acc():
        s0[...] += ps0
        s1[...] += ps1
        cnt[...] += pc

    @pl.when(j == nb - 1)
    def _fin():
        v = jnp.dot(s0[...], wv_ref[0:DH, :], preferred_element_type=_f32)
        v += jnp.dot(s1[...], wv_ref[DH:2 * DH, :], preferred_element_type=_f32)
        v = v / jnp.maximum(cnt[...], 1.0) + bv_ref[...]
        val_ref[...] = jnp.tanh(v)


def _head(h3, batch2, Wh, bh2, Wv, bv2):
    return pl.pallas_call(
        _head_body,
        grid=(N_NODES // NB,),
        in_specs=[
            pl.BlockSpec((2, NB, DH), lambda j: (0, j, 0)),
            pl.BlockSpec((NB, 1), lambda j: (j, 0)),
            pl.BlockSpec((D, 1), lambda j: (0, 0)),
            pl.BlockSpec((1, 1), lambda j: (0, 0)),
            pl.BlockSpec((D, 1), lambda j: (0, 0)),
            pl.BlockSpec((1, 1), lambda j: (0, 0)),
        ],
        out_specs=[
            pl.BlockSpec((NB, 1), lambda j: (j, 0)),
            pl.BlockSpec((NG, 1), lambda j: (0, 0)),
        ],
        out_shape=[
            jax.ShapeDtypeStruct((N_NODES, 1), _f32),
            jax.ShapeDtypeStruct((NG, 1), _f32),
        ],
        scratch_shapes=[
            pltpu.VMEM((NG, DH), _f32),
            pltpu.VMEM((NG, DH), _f32),
            pltpu.VMEM((NG, 1), _f32),
        ],
    )(h3, batch2, Wh, bh2, Wv, bv2)


def kernel(x, edge_index, edge_attr, batch,
           We0, be0, W0, b0, We1, be1, W1, b1, We2, be2, W2, b2,
           Wh, bh, Wv, bv):
    src = edge_index[0].astype(jnp.int32)
    dst = edge_index[1].astype(jnp.int32)
    batch2 = batch.astype(jnp.int32).reshape(N_NODES, 1)

    h = jnp.concatenate([x[:, :DH], x[:, DH:]], axis=0)
    layers = [(We0, be0, W0, b0), (We1, be1, W1, b1), (We2, be2, W2, b2)]
    for We, be, W, b in layers:
        e_i32 = _edge_mlp(edge_attr, We, be.reshape(1, D))
        agg = _sc_edge(h, e_i32, src, dst)
        h = _node_update(h.reshape(2, N_NODES, DH),
                         agg.reshape(2, N_NODES, DH), W, b.reshape(1, D))

    logits2, value2 = _head(h.reshape(2, N_NODES, DH), batch2,
                            Wh, bh.reshape(1, 1), Wv, bv.reshape(1, 1))
    return logits2.ravel(), value2.ravel()


# speedup vs baseline: 1.1162x; 1.1162x over previous
import functools

import jax
import jax.numpy as jnp
from jax import lax
from jax.experimental import pallas as pl
from jax.experimental.pallas import tpu as pltpu
from jax.experimental.pallas import tpu_sc as plsc

N_NODES = 10000
N_EDGES = 160000
D = 256
DH = 128
NG = 64

EB = 2000
NB = 2000
SCB = 80
N_SUB = 16
EPT = N_EDGES // N_SUB
NWR = 10
RPT = N_NODES // NWR
ZROWS = 40
WROWS = 200

_f32 = jnp.float32


def _edge_body(ea_ref, we_ref, be_ref, out_ref):
    acc = jnp.dot(ea_ref[...], we_ref[...], preferred_element_type=_f32)
    eb = jnp.maximum(acc + be_ref[...], 0.0)
    ebp = eb.reshape(EB // 2, 2, DH)
    a = ebp[:, 0, :].astype(jnp.bfloat16).astype(_f32)
    b = ebp[:, 1, :].astype(jnp.bfloat16).astype(_f32)
    ai = lax.bitcast_convert_type(a, jnp.int32)
    bi = lax.bitcast_convert_type(b, jnp.int32)
    out_ref[...] = lax.bitwise_or(
        lax.bitwise_and(bi, jnp.int32(-65536)),
        lax.shift_right_logical(ai, 16))


def _edge_mlp(edge_attr, We, be2):
    nblk = N_EDGES // EB
    return pl.pallas_call(
        _edge_body,
        grid=(2, nblk),
        in_specs=[
            pl.BlockSpec((EB, 16), lambda c, j: (j, 0)),
            pl.BlockSpec((16, DH), lambda c, j: (0, c)),
            pl.BlockSpec((1, DH), lambda c, j: (0, c)),
        ],
        out_specs=pl.BlockSpec((EB // 2, DH),
                               lambda c, j, _n=nblk: (c * _n + j, 0)),
        out_shape=jax.ShapeDtypeStruct((N_EDGES, DH), jnp.int32),
    )(edge_attr, We, be2)


_sc_mesh = plsc.VectorSubcoreMesh(core_axis_name="c", subcore_axis_name="s")


IB = 25
NBATCH = EPT // (IB * SCB)


@functools.partial(
    pl.kernel,
    out_type=jax.ShapeDtypeStruct((2 * N_NODES, DH), _f32),
    mesh=_sc_mesh,
    scratch_types=[
        pltpu.VMEM((IB * SCB,), jnp.int32),
        pltpu.VMEM((IB * SCB,), jnp.int32),
        pltpu.VMEM((SCB,), jnp.int32),
        pltpu.VMEM((SCB,), jnp.int32),
        pltpu.VMEM((SCB, DH), _f32),
        pltpu.VMEM((SCB, DH), _f32),
        pltpu.VMEM((SCB // 2, DH), jnp.int32),
        pltpu.VMEM((SCB // 2, DH), jnp.int32),
        pltpu.VMEM((ZROWS, DH), _f32),
        pltpu.VMEM_SHARED((N_NODES, DH), _f32),
        pltpu.SemaphoreType.DMA,
        pltpu.SemaphoreType.DMA,
    ],
)
def _sc_edge(h_hbm, e_hbm, src_hbm, dst_hbm, out_hbm,
             idx_sbig, idx_dbig, idx_d0, idx_d1, g0, g1, e0, e1,
             zero_v, shared, sem0, sem1):
    c = lax.axis_index("c")
    s = lax.axis_index("s")
    idx_d = (idx_d0, idx_d1)
    gath = (g0, g1)
    e_v = (e0, e1)
    sem = (sem0, sem1)

    def zrow(r, carry):
        for t in range(DH // 16):
            zero_v[r, pl.ds(t * 16, 16)] = jnp.zeros((16,), _f32)
        return carry

    @pl.when(s < NWR)
    def _zero():
        lax.fori_loop(0, ZROWS, zrow, 0)
        for i in range(RPT // ZROWS):
            pltpu.sync_copy(zero_v, shared.at[pl.ds(s * RPT + i * ZROWS, ZROWS)])

    plsc.subcore_barrier()

    ebase = s * EPT

    def batch(t, carry):
        boff = ebase + t * (IB * SCB)
        eoff2 = (c * (N_EDGES // 2) + s * (EPT // 2)
                 + t * (IB * SCB // 2))
        pltpu.sync_copy(src_hbm.at[pl.ds(boff, IB * SCB)], idx_sbig)
        pltpu.sync_copy(dst_hbm.at[pl.ds(boff, IB * SCB)], idx_dbig)

        def adj(i, ac):
            sl = pl.ds(i * 16, 16)
            idx_sbig[sl] = idx_sbig[sl] + c * N_NODES
            return ac

        lax.fori_loop(0, IB * SCB // 16, adj, 0)

        def start(k, b):
            for t in range(SCB // 16):
                idx_d[b][pl.ds(t * 16, 16)] = idx_dbig[pl.ds(k * SCB + t * 16, 16)]
            pltpu.async_copy(h_hbm.at[idx_sbig.at[pl.ds(k * SCB, SCB)]],
                             gath[b], sem[b])
            pltpu.async_copy(
                e_hbm.at[pl.ds(eoff2 + k * (SCB // 2), SCB // 2)],
                e_v[b], sem[b])

        def finish(k, b):
            pltpu.make_async_copy(h_hbm.at[idx_sbig.at[pl.ds(k * SCB, SCB)]],
                                  gath[b], sem[b]).wait()
            pltpu.make_async_copy(
                e_hbm.at[pl.ds(eoff2 + k * (SCB // 2), SCB // 2)],
                e_v[b], sem[b]).wait()

            @plsc.parallel_loop(0, SCB // 2, unroll=2)
            def _row(r2):
                for g in range(DH // 16):
                    sl = pl.ds(g * 16, 16)
                    w = e_v[b][r2, sl]
                    lo = lax.bitcast_convert_type(
                        lax.shift_left(w, 16), _f32)
                    hi = lax.bitcast_convert_type(
                        lax.bitwise_and(w, jnp.int32(-65536)), _f32)
                    gath[b][2 * r2, sl] = jnp.maximum(
                        gath[b][2 * r2, sl] + lo, 0.0)
                    gath[b][2 * r2 + 1, sl] = jnp.maximum(
                        gath[b][2 * r2 + 1, sl] + hi, 0.0)

            pltpu.sync_copy(gath[b], shared.at[idx_d[b]], add=True)

        start(0, 0)
        for k in range(IB - 1):
            start(k + 1, (k + 1) % 2)
            finish(k, k % 2)
        finish(IB - 1, (IB - 1) % 2)
        return carry

    lax.fori_loop(0, NBATCH, batch, 0)
    plsc.subcore_barrier()

    @pl.when(s < NWR)
    def _writeout():
        for i in range(RPT // WROWS):
            rb = s * RPT + i * WROWS
            pltpu.sync_copy(shared.at[pl.ds(rb, WROWS)],
                            out_hbm.at[pl.ds(c * N_NODES + rb, WROWS)])


def _node_body(h_ref, agg_ref, w_ref, b_ref, out_ref):
    hp0 = h_ref[0] + agg_ref[0]
    hp1 = h_ref[1] + agg_ref[1]
    acc = jnp.dot(hp0, w_ref[0:DH, :], preferred_element_type=_f32)
    acc += jnp.dot(hp1, w_ref[DH:2 * DH, :], preferred_element_type=_f32)
    out_ref[...] = jnp.maximum(acc + b_ref[...], 0.0)


def _node_update(h3, agg3, W, b2):
    nblk = N_NODES // NB
    return pl.pallas_call(
        _node_body,
        grid=(2, nblk),
        in_specs=[
            pl.BlockSpec((2, NB, DH), lambda c2, j: (0, j, 0)),
            pl.BlockSpec((2, NB, DH), lambda c2, j: (0, j, 0)),
            pl.BlockSpec((D, DH), lambda c2, j: (0, c2)),
            pl.BlockSpec((1, DH), lambda c2, j: (0, c2)),
        ],
        out_specs=pl.BlockSpec((NB, DH), lambda c2, j, _n=nblk: (c2 * _n + j, 0)),
        out_shape=jax.ShapeDtypeStruct((2 * N_NODES, DH), _f32),
    )(h3, agg3, W, b2)


def _head_body(h_ref, bat_ref, wh_ref, bh_ref, wv_ref, bv_ref,
               log_ref, val_ref, s0, s1, cnt):
    j = pl.program_id(0)
    nb = pl.num_programs(0)
    h0 = h_ref[0]
    h1 = h_ref[1]
    lg = jnp.dot(h0, wh_ref[0:DH, :], preferred_element_type=_f32)
    lg += jnp.dot(h1, wh_ref[DH:2 * DH, :], preferred_element_type=_f32)
    log_ref[...] = lg + bh_ref[...]

    onehot = (bat_ref[...] == lax.broadcasted_iota(jnp.int32, (1, NG), 1))
    onehot = onehot.astype(_f32)
    dn = (((0,), (0,)), ((), ()))
    ps0 = lax.dot_general(onehot, h0, dn, preferred_element_type=_f32)
    ps1 = lax.dot_general(onehot, h1, dn, preferred_element_type=_f32)
    pc = lax.dot_general(onehot, jnp.ones((NB, 1), _f32), dn,
                         preferred_element_type=_f32)

    @pl.when(j == 0)
    def _init():
        s0[...] = ps0
        s1[...] = ps1
        cnt[...] = pc

    @pl.when(j > 0)
    def _acc():
        s0[...] += ps0
        s1[...] += ps1
        cnt[...] += pc

    @pl.when(j == nb - 1)
    def _fin():
        v = jnp.dot(s0[...], wv_ref[0:DH, :], preferred_element_type=_f32)
        v += jnp.dot(s1[...], wv_ref[DH:2 * DH, :], preferred_element_type=_f32)
        v = v / jnp.maximum(cnt[...], 1.0) + bv_ref[...]
        val_ref[...] = jnp.tanh(v)


def _head(h3, batch2, Wh, bh2, Wv, bv2):
    return pl.pallas_call(
        _head_body,
        grid=(N_NODES // NB,),
        in_specs=[
            pl.BlockSpec((2, NB, DH), lambda j: (0, j, 0)),
            pl.BlockSpec((NB, 1), lambda j: (j, 0)),
            pl.BlockSpec((D, 1), lambda j: (0, 0)),
            pl.BlockSpec((1, 1), lambda j: (0, 0)),
            pl.BlockSpec((D, 1), lambda j: (0, 0)),
            pl.BlockSpec((1, 1), lambda j: (0, 0)),
        ],
        out_specs=[
            pl.BlockSpec((NB, 1), lambda j: (j, 0)),
            pl.BlockSpec((NG, 1), lambda j: (0, 0)),
        ],
        out_shape=[
            jax.ShapeDtypeStruct((N_NODES, 1), _f32),
            jax.ShapeDtypeStruct((NG, 1), _f32),
        ],
        scratch_shapes=[
            pltpu.VMEM((NG, DH), _f32),
            pltpu.VMEM((NG, DH), _f32),
            pltpu.VMEM((NG, 1), _f32),
        ],
    )(h3, batch2, Wh, bh2, Wv, bv2)


def kernel(x, edge_index, edge_attr, batch,
           We0, be0, W0, b0, We1, be1, W1, b1, We2, be2, W2, b2,
           Wh, bh, Wv, bv):
    src = edge_index[0].astype(jnp.int32)
    dst = edge_index[1].astype(jnp.int32)
    batch2 = batch.astype(jnp.int32).reshape(N_NODES, 1)

    h = jnp.concatenate([x[:, :DH], x[:, DH:]], axis=0)
    layers = [(We0, be0, W0, b0), (We1, be1, W1, b1), (We2, be2, W2, b2)]
    for We, be, W, b in layers:
        e_i32 = _edge_mlp(edge_attr, We, be.reshape(1, D))
        agg = _sc_edge(h, e_i32, src, dst)
        h = _node_update(h.reshape(2, N_NODES, DH),
                         agg.reshape(2, N_NODES, DH), W, b.reshape(1, D))

    logits2, value2 = _head(h.reshape(2, N_NODES, DH), batch2,
                            Wh, bh.reshape(1, 1), Wv, bv.reshape(1, 1))
    return logits2.ravel(), value2.ravel()


# speedup vs baseline: 1.1163x; 1.0001x over previous
import functools

import jax
import jax.numpy as jnp
from jax import lax
from jax.experimental import pallas as pl
from jax.experimental.pallas import tpu as pltpu
from jax.experimental.pallas import tpu_sc as plsc

N_NODES = 10000
N_EDGES = 160000
D = 256
DH = 128
NG = 64

EB = 2000
NB = 2000
SCB = 80
N_SUB = 16
EPT = N_EDGES // N_SUB
NWR = 10
RPT = N_NODES // NWR
ZROWS = 40
WROWS = 200

_f32 = jnp.float32


def _edge_body(ea_ref, we_ref, be_ref, out_ref):
    acc = jnp.dot(ea_ref[...], we_ref[...], preferred_element_type=_f32)
    eb = jnp.maximum(acc + be_ref[...], 0.0)
    ebp = eb.reshape(EB // 2, 2, DH)
    a = ebp[:, 0, :].astype(jnp.bfloat16).astype(_f32)
    b = ebp[:, 1, :].astype(jnp.bfloat16).astype(_f32)
    ai = lax.bitcast_convert_type(a, jnp.int32)
    bi = lax.bitcast_convert_type(b, jnp.int32)
    out_ref[...] = lax.bitwise_or(
        lax.bitwise_and(bi, jnp.int32(-65536)),
        lax.shift_right_logical(ai, 16))


def _edge_mlp(edge_attr, We, be2):
    nblk = N_EDGES // EB
    return pl.pallas_call(
        _edge_body,
        grid=(2, nblk),
        in_specs=[
            pl.BlockSpec((EB, 16), lambda c, j: (j, 0)),
            pl.BlockSpec((16, DH), lambda c, j: (0, c)),
            pl.BlockSpec((1, DH), lambda c, j: (0, c)),
        ],
        out_specs=pl.BlockSpec((EB // 2, DH),
                               lambda c, j, _n=nblk: (c * _n + j, 0)),
        out_shape=jax.ShapeDtypeStruct((N_EDGES, DH), jnp.int32),
    )(edge_attr, We, be2)


_sc_mesh = plsc.VectorSubcoreMesh(core_axis_name="c", subcore_axis_name="s")


IB = 25
NBATCH = EPT // (IB * SCB)


@functools.partial(
    pl.kernel,
    out_type=jax.ShapeDtypeStruct((2 * N_NODES, DH), _f32),
    mesh=_sc_mesh,
    scratch_types=[
        pltpu.VMEM((IB * SCB,), jnp.int32),
        pltpu.VMEM((IB * SCB,), jnp.int32),
        pltpu.VMEM((SCB,), jnp.int32),
        pltpu.VMEM((SCB,), jnp.int32),
        pltpu.VMEM((SCB, DH), _f32),
        pltpu.VMEM((SCB, DH), _f32),
        pltpu.VMEM((SCB // 2, DH), jnp.int32),
        pltpu.VMEM((SCB // 2, DH), jnp.int32),
        pltpu.VMEM((ZROWS, DH), _f32),
        pltpu.VMEM_SHARED((N_NODES, DH), _f32),
        pltpu.SemaphoreType.DMA,
        pltpu.SemaphoreType.DMA,
    ],
)
def _sc_edge(h_hbm, e_hbm, src_hbm, dst_hbm, out_hbm,
             idx_sbig, idx_dbig, idx_d0, idx_d1, g0, g1, e0, e1,
             zero_v, shared, sem0, sem1):
    c = lax.axis_index("c")
    s = lax.axis_index("s")
    idx_d = (idx_d0, idx_d1)
    gath = (g0, g1)
    e_v = (e0, e1)
    sem = (sem0, sem1)

    def zrow(r, carry):
        for t in range(DH // 16):
            zero_v[r, pl.ds(t * 16, 16)] = jnp.zeros((16,), _f32)
        return carry

    @pl.when(s < NWR)
    def _zero():
        lax.fori_loop(0, ZROWS, zrow, 0)
        for i in range(RPT // ZROWS):
            pltpu.sync_copy(zero_v, shared.at[pl.ds(s * RPT + i * ZROWS, ZROWS)])

    plsc.subcore_barrier()

    ebase = s * EPT

    def batch(t, carry):
        boff = ebase + t * (IB * SCB)
        eoff2 = (c * (N_EDGES // 2) + s * (EPT // 2)
                 + t * (IB * SCB // 2))
        pltpu.sync_copy(src_hbm.at[pl.ds(boff, IB * SCB)], idx_sbig)
        pltpu.sync_copy(dst_hbm.at[pl.ds(boff, IB * SCB)], idx_dbig)

        def adj(i, ac):
            sl = pl.ds(i * 16, 16)
            idx_sbig[sl] = idx_sbig[sl] + c * N_NODES
            return ac

        lax.fori_loop(0, IB * SCB // 16, adj, 0)

        def start(k, b):
            for t in range(SCB // 16):
                idx_d[b][pl.ds(t * 16, 16)] = idx_dbig[pl.ds(k * SCB + t * 16, 16)]
            pltpu.async_copy(h_hbm.at[idx_sbig.at[pl.ds(k * SCB, SCB)]],
                             gath[b], sem[b])
            pltpu.async_copy(
                e_hbm.at[pl.ds(eoff2 + k * (SCB // 2), SCB // 2)],
                e_v[b], sem[b])

        def finish(k, b):
            pltpu.make_async_copy(h_hbm.at[idx_sbig.at[pl.ds(k * SCB, SCB)]],
                                  gath[b], sem[b]).wait()
            pltpu.make_async_copy(
                e_hbm.at[pl.ds(eoff2 + k * (SCB // 2), SCB // 2)],
                e_v[b], sem[b]).wait()

            @plsc.parallel_loop(0, SCB // 2, unroll=2)
            def _row(r2):
                for g in range(DH // 16):
                    sl = pl.ds(g * 16, 16)
                    w = e_v[b][r2, sl]
                    lo = lax.bitcast_convert_type(
                        lax.shift_left(w, 16), _f32)
                    hi = lax.bitcast_convert_type(
                        lax.bitwise_and(w, jnp.int32(-65536)), _f32)
                    gath[b][2 * r2, sl] = jnp.maximum(
                        gath[b][2 * r2, sl] + lo, 0.0)
                    gath[b][2 * r2 + 1, sl] = jnp.maximum(
                        gath[b][2 * r2 + 1, sl] + hi, 0.0)

            pltpu.sync_copy(gath[b], shared.at[idx_d[b]], add=True)

        start(0, 0)
        for k in range(IB - 1):
            start(k + 1, (k + 1) % 2)
            finish(k, k % 2)
        finish(IB - 1, (IB - 1) % 2)
        return carry

    lax.fori_loop(0, NBATCH, batch, 0)
    plsc.subcore_barrier()

    @pl.when(s < NWR)
    def _writeout():
        for i in range(RPT // WROWS):
            rb = s * RPT + i * WROWS
            pltpu.sync_copy(shared.at[pl.ds(rb, WROWS)],
                            out_hbm.at[pl.ds(c * N_NODES + rb, WROWS)])


def _node_body(h_ref, agg_ref, w_ref, b_ref, out_ref):
    hp0 = h_ref[0] + agg_ref[0]
    hp1 = h_ref[1] + agg_ref[1]
    acc = jnp.dot(hp0, w_ref[0:DH, :], preferred_element_type=_f32)
    acc += jnp.dot(hp1, w_ref[DH:2 * DH, :], preferred_element_type=_f32)
    out_ref[...] = jnp.maximum(acc + b_ref[...], 0.0)


def _node_update(h3, agg3, W, b2):
    nblk = N_NODES // NB
    return pl.pallas_call(
        _node_body,
        grid=(2, nblk),
        in_specs=[
            pl.BlockSpec((2, NB, DH), lambda c2, j: (0, j, 0)),
            pl.BlockSpec((2, NB, DH), lambda c2, j: (0, j, 0)),
            pl.BlockSpec((D, DH), lambda c2, j: (0, c2)),
            pl.BlockSpec((1, DH), lambda c2, j: (0, c2)),
        ],
        out_specs=pl.BlockSpec((NB, DH), lambda c2, j, _n=nblk: (c2 * _n + j, 0)),
        out_shape=jax.ShapeDtypeStruct((2 * N_NODES, DH), _f32),
    )(h3, agg3, W, b2)


def _head_body(h_ref, bat_ref, wh_ref, bh_ref, wv_ref, bv_ref,
               log_ref, val_ref, s0, s1, cnt):
    j = pl.program_id(0)
    nb = pl.num_programs(0)
    h0 = h_ref[0]
    h1 = h_ref[1]
    lg = jnp.dot(h0, wh_ref[0:DH, :], preferred_element_type=_f32)
    lg += jnp.dot(h1, wh_ref[DH:2 * DH, :], preferred_element_type=_f32)
    log_ref[...] = lg + bh_ref[...]

    onehot = (bat_ref[...] == lax.broadcasted_iota(jnp.int32, (1, NG), 1))
    onehot = onehot.astype(_f32)
    dn = (((0,), (0,)), ((), ()))
    ps0 = lax.dot_general(onehot, h0, dn, preferred_element_type=_f32)
    ps1 = lax.dot_general(onehot, h1, dn, preferred_element_type=_f32)
    pc = lax.dot_general(onehot, jnp.ones((NB, 1), _f32), dn,
                         preferred_element_type=_f32)

    @pl.when(j == 0)
    def _init():
        s0[...] = ps0
        s1[...] = ps1
        cnt[...] = pc

    @pl.when(j > 0)
    def _acc():
        s0[...] += ps0
        s1[...] += ps1
        cnt[...] += pc

    @pl.when(j == nb - 1)
    def _fin():
        v = jnp.dot(s0[...], wv_ref[0:DH, :], preferred_element_type=_f32)
        v += jnp.dot(s1[...], wv_ref[DH:2 * DH, :], preferred_element_type=_f32)
        v = v / jnp.maximum(cnt[...], 1.0) + bv_ref[...]
        val_ref[...] = jnp.tanh(v)


def _head(h3, batch2, Wh, bh2, Wv, bv2):
    return pl.pallas_call(
        _head_body,
        grid=(N_NODES // NB,),
        in_specs=[
            pl.BlockSpec((2, NB, DH), lambda j: (0, j, 0)),
            pl.BlockSpec((NB, 1), lambda j: (j, 0)),
            pl.BlockSpec((D, 1), lambda j: (0, 0)),
            pl.BlockSpec((1, 1), lambda j: (0, 0)),
            pl.BlockSpec((D, 1), lambda j: (0, 0)),
            pl.BlockSpec((1, 1), lambda j: (0, 0)),
        ],
        out_specs=[
            pl.BlockSpec((NB, 1), lambda j: (j, 0)),
            pl.BlockSpec((NG, 1), lambda j: (0, 0)),
        ],
        out_shape=[
            jax.ShapeDtypeStruct((N_NODES, 1), _f32),
            jax.ShapeDtypeStruct((NG, 1), _f32),
        ],
        scratch_shapes=[
            pltpu.VMEM((NG, DH), _f32),
            pltpu.VMEM((NG, DH), _f32),
            pltpu.VMEM((NG, 1), _f32),
        ],
    )(h3, batch2, Wh, bh2, Wv, bv2)


def kernel(x, edge_index, edge_attr, batch,
           We0, be0, W0, b0, We1, be1, W1, b1, We2, be2, W2, b2,
           Wh, bh, Wv, bv):
    src = edge_index[0].astype(jnp.int32)
    dst = edge_index[1].astype(jnp.int32)
    batch2 = batch.astype(jnp.int32).reshape(N_NODES, 1)

    h = jnp.concatenate([x[:, :DH], x[:, DH:]], axis=0)
    layers = [(We0, be0, W0, b0), (We1, be1, W1, b1), (We2, be2, W2, b2)]
    e_all = [_edge_mlp(edge_attr, We, be.reshape(1, D))
             for We, be, _, _ in layers]
    for (We, be, W, b), e_i32 in zip(layers, e_all):
        agg = _sc_edge(h, e_i32, src, dst)
        h = _node_update(h.reshape(2, N_NODES, DH),
                         agg.reshape(2, N_NODES, DH), W, b.reshape(1, D))

    logits2, value2 = _head(h.reshape(2, N_NODES, DH), batch2,
                            Wh, bh.reshape(1, 1), Wv, bv.reshape(1, 1))
    return logits2.ravel(), value2.ravel()


# speedup vs baseline: 1.1293x; 1.0116x over previous
import functools

import jax
import jax.numpy as jnp
from jax import lax
from jax.experimental import pallas as pl
from jax.experimental.pallas import tpu as pltpu
from jax.experimental.pallas import tpu_sc as plsc

N_NODES = 10000
N_EDGES = 160000
D = 256
DH = 128
NG = 64

EB = 2000
NB = 2000
SCB = 80
N_SUB = 16
EPT = N_EDGES // N_SUB
NWR = 10
RPT = N_NODES // NWR
ZROWS = 40
WROWS = 200

_f32 = jnp.float32


def _edge_body(ea_ref, we_ref, be_ref, out_ref):
    acc = jnp.dot(ea_ref[...], we_ref[...], preferred_element_type=_f32)
    out_ref[...] = jnp.maximum(acc + be_ref[...], 0.0)


def _edge_mlp(edge_attr, We, be2):
    nblk = N_EDGES // EB
    return pl.pallas_call(
        _edge_body,
        grid=(2, nblk),
        in_specs=[
            pl.BlockSpec((EB, 16), lambda c, j: (j, 0)),
            pl.BlockSpec((16, DH), lambda c, j: (0, c)),
            pl.BlockSpec((1, DH), lambda c, j: (0, c)),
        ],
        out_specs=pl.BlockSpec((EB, DH), lambda c, j, _n=nblk: (c * _n + j, 0)),
        out_shape=jax.ShapeDtypeStruct((2 * N_EDGES, DH), _f32),
    )(edge_attr, We, be2)


_sc_mesh = plsc.VectorSubcoreMesh(core_axis_name="c", subcore_axis_name="s")


IB = 25
NBATCH = EPT // (IB * SCB)


@functools.partial(
    pl.kernel,
    out_type=jax.ShapeDtypeStruct((2 * N_NODES, DH), _f32),
    mesh=_sc_mesh,
    scratch_types=[
        pltpu.VMEM((IB * SCB,), jnp.int32),
        pltpu.VMEM((IB * SCB,), jnp.int32),
        pltpu.VMEM((SCB,), jnp.int32),
        pltpu.VMEM((SCB,), jnp.int32),
        pltpu.VMEM((SCB, DH), _f32),
        pltpu.VMEM((SCB, DH), _f32),
        pltpu.VMEM((SCB, DH), _f32),
        pltpu.VMEM((SCB, DH), _f32),
        pltpu.VMEM((ZROWS, DH), _f32),
        pltpu.VMEM_SHARED((N_NODES, DH), _f32),
        pltpu.SemaphoreType.DMA,
        pltpu.SemaphoreType.DMA,
    ],
)
def _sc_edge(h_hbm, e_hbm, src_hbm, dst_hbm, out_hbm,
             idx_sbig, idx_dbig, idx_d0, idx_d1, g0, g1, e0, e1,
             zero_v, shared, sem0, sem1):
    c = lax.axis_index("c")
    s = lax.axis_index("s")
    idx_d = (idx_d0, idx_d1)
    gath = (g0, g1)
    e_v = (e0, e1)
    sem = (sem0, sem1)

    def zrow(r, carry):
        for t in range(DH // 16):
            zero_v[r, pl.ds(t * 16, 16)] = jnp.zeros((16,), _f32)
        return carry

    @pl.when(s < NWR)
    def _zero():
        lax.fori_loop(0, ZROWS, zrow, 0)
        for i in range(RPT // ZROWS):
            pltpu.sync_copy(zero_v, shared.at[pl.ds(s * RPT + i * ZROWS, ZROWS)])

    plsc.subcore_barrier()

    ebase = s * EPT

    def batch(t, carry):
        boff = ebase + t * (IB * SCB)
        pltpu.sync_copy(src_hbm.at[pl.ds(boff, IB * SCB)], idx_sbig)
        pltpu.sync_copy(dst_hbm.at[pl.ds(boff, IB * SCB)], idx_dbig)

        def adj(i, ac):
            sl = pl.ds(i * 16, 16)
            idx_sbig[sl] = idx_sbig[sl] + c * N_NODES
            return ac

        lax.fori_loop(0, IB * SCB // 16, adj, 0)

        def start(k, b):
            for t in range(SCB // 16):
                idx_d[b][pl.ds(t * 16, 16)] = idx_dbig[pl.ds(k * SCB + t * 16, 16)]
            pltpu.async_copy(h_hbm.at[idx_sbig.at[pl.ds(k * SCB, SCB)]],
                             gath[b], sem[b])
            pltpu.async_copy(
                e_hbm.at[pl.ds(c * N_EDGES + boff + k * SCB, SCB)],
                e_v[b], sem[b])

        def finish(k, b):
            pltpu.make_async_copy(h_hbm.at[idx_sbig.at[pl.ds(k * SCB, SCB)]],
                                  gath[b], sem[b]).wait()
            pltpu.make_async_copy(
                e_hbm.at[pl.ds(c * N_EDGES + boff + k * SCB, SCB)],
                e_v[b], sem[b]).wait()

            @plsc.parallel_loop(0, SCB, unroll=4)
            def _row(r):
                for g in range(DH // 16):
                    sl = pl.ds(g * 16, 16)
                    gath[b][r, sl] = jnp.maximum(
                        gath[b][r, sl] + e_v[b][r, sl], 0.0)

            pltpu.sync_copy(gath[b], shared.at[idx_d[b]], add=True)

        start(0, 0)
        for k in range(IB - 1):
            start(k + 1, (k + 1) % 2)
            finish(k, k % 2)
        finish(IB - 1, (IB - 1) % 2)
        return carry

    lax.fori_loop(0, NBATCH, batch, 0)
    plsc.subcore_barrier()

    @pl.when(s < NWR)
    def _writeout():
        for i in range(RPT // WROWS):
            rb = s * RPT + i * WROWS
            pltpu.sync_copy(shared.at[pl.ds(rb, WROWS)],
                            out_hbm.at[pl.ds(c * N_NODES + rb, WROWS)])


def _node_body(h_ref, agg_ref, w_ref, b_ref, out_ref):
    hp0 = h_ref[0] + agg_ref[0]
    hp1 = h_ref[1] + agg_ref[1]
    acc = jnp.dot(hp0, w_ref[0:DH, :], preferred_element_type=_f32)
    acc += jnp.dot(hp1, w_ref[DH:2 * DH, :], preferred_element_type=_f32)
    out_ref[...] = jnp.maximum(acc + b_ref[...], 0.0)


def _node_update(h3, agg3, W, b2):
    nblk = N_NODES // NB
    return pl.pallas_call(
        _node_body,
        grid=(2, nblk),
        in_specs=[
            pl.BlockSpec((2, NB, DH), lambda c2, j: (0, j, 0)),
            pl.BlockSpec((2, NB, DH), lambda c2, j: (0, j, 0)),
            pl.BlockSpec((D, DH), lambda c2, j: (0, c2)),
            pl.BlockSpec((1, DH), lambda c2, j: (0, c2)),
        ],
        out_specs=pl.BlockSpec((NB, DH), lambda c2, j, _n=nblk: (c2 * _n + j, 0)),
        out_shape=jax.ShapeDtypeStruct((2 * N_NODES, DH), _f32),
    )(h3, agg3, W, b2)


def _head_body(h_ref, bat_ref, wh_ref, bh_ref, wv_ref, bv_ref,
               log_ref, val_ref, s0, s1, cnt):
    j = pl.program_id(0)
    nb = pl.num_programs(0)
    h0 = h_ref[0]
    h1 = h_ref[1]
    lg = jnp.dot(h0, wh_ref[0:DH, :], preferred_element_type=_f32)
    lg += jnp.dot(h1, wh_ref[DH:2 * DH, :], preferred_element_type=_f32)
    log_ref[...] = lg + bh_ref[...]

    onehot = (bat_ref[...] == lax.broadcasted_iota(jnp.int32, (1, NG), 1))
    onehot = onehot.astype(_f32)
    dn = (((0,), (0,)), ((), ()))
    ps0 = lax.dot_general(onehot, h0, dn, preferred_element_type=_f32)
    ps1 = lax.dot_general(onehot, h1, dn, preferred_element_type=_f32)
    pc = lax.dot_general(onehot, jnp.ones((NB, 1), _f32), dn,
                         preferred_element_type=_f32)

    @pl.when(j == 0)
    def _init():
        s0[...] = ps0
        s1[...] = ps1
        cnt[...] = pc

    @pl.when(j > 0)
    def _acc():
        s0[...] += ps0
        s1[...] += ps1
        cnt[...] += pc

    @pl.when(j == nb - 1)
    def _fin():
        v = jnp.dot(s0[...], wv_ref[0:DH, :], preferred_element_type=_f32)
        v += jnp.dot(s1[...], wv_ref[DH:2 * DH, :], preferred_element_type=_f32)
        v = v / jnp.maximum(cnt[...], 1.0) + bv_ref[...]
        val_ref[...] = jnp.tanh(v)


def _head(h3, batch2, Wh, bh2, Wv, bv2):
    return pl.pallas_call(
        _head_body,
        grid=(N_NODES // NB,),
        in_specs=[
            pl.BlockSpec((2, NB, DH), lambda j: (0, j, 0)),
            pl.BlockSpec((NB, 1), lambda j: (j, 0)),
            pl.BlockSpec((D, 1), lambda j: (0, 0)),
            pl.BlockSpec((1, 1), lambda j: (0, 0)),
            pl.BlockSpec((D, 1), lambda j: (0, 0)),
            pl.BlockSpec((1, 1), lambda j: (0, 0)),
        ],
        out_specs=[
            pl.BlockSpec((NB, 1), lambda j: (j, 0)),
            pl.BlockSpec((NG, 1), lambda j: (0, 0)),
        ],
        out_shape=[
            jax.ShapeDtypeStruct((N_NODES, 1), _f32),
            jax.ShapeDtypeStruct((NG, 1), _f32),
        ],
        scratch_shapes=[
            pltpu.VMEM((NG, DH), _f32),
            pltpu.VMEM((NG, DH), _f32),
            pltpu.VMEM((NG, 1), _f32),
        ],
    )(h3, batch2, Wh, bh2, Wv, bv2)


def kernel(x, edge_index, edge_attr, batch,
           We0, be0, W0, b0, We1, be1, W1, b1, We2, be2, W2, b2,
           Wh, bh, Wv, bv):
    src = edge_index[0].astype(jnp.int32)
    dst = edge_index[1].astype(jnp.int32)
    batch2 = batch.astype(jnp.int32).reshape(N_NODES, 1)

    h = jnp.concatenate([x[:, :DH], x[:, DH:]], axis=0)
    layers = [(We0, be0, W0, b0), (We1, be1, W1, b1), (We2, be2, W2, b2)]
    e_all = [_edge_mlp(edge_attr, We, be.reshape(1, D))
             for We, be, _, _ in layers]
    for (We, be, W, b), e_i32 in zip(layers, e_all):
        agg = _sc_edge(h, e_i32, src, dst)
        h = _node_update(h.reshape(2, N_NODES, DH),
                         agg.reshape(2, N_NODES, DH), W, b.reshape(1, D))

    logits2, value2 = _head(h.reshape(2, N_NODES, DH), batch2,
                            Wh, bh.reshape(1, 1), Wv, bv.reshape(1, 1))
    return logits2.ravel(), value2.ravel()
